# baseline (speedup 1.0000x reference)
import jax
import jax.numpy as jnp
from jax.experimental import pallas as pl
from jax.experimental.pallas import tpu as pltpu

_RSM_K = 35
_PFM_K = 31
_F8 = jnp.float8_e4m3fn
_BLK = 256


def _su(x, d):
    return jnp.concatenate([x[d:, :], jnp.zeros((d, x.shape[1]), x.dtype)], axis=0)


def _sd(x, d):
    return jnp.concatenate([jnp.zeros((d, x.shape[1]), x.dtype), x[:-d, :]], axis=0)


def _kspan(j, w, r):
    k0 = max(0, ((j * _BLK - r) // 128) * 128)
    k1 = min(w, ((j * _BLK + _BLK + r + 127) // 128) * 128)
    return k0, k1


def _one_image(xf, a35b_ref, a35q_ref, a31q_ref, rsm_ref, pfm_ref, g):
    w = xf.shape[1]
    nb = w // _BLK if w % _BLK == 0 else 1
    blk = w // nb
    r17, r15 = _RSM_K // 2, _PFM_K // 2

    xq = xf.astype(_F8)
    rows = []
    for i in range(nb):
        k0, k1 = _kspan(i, w, r17) if nb > 1 else (0, w)
        rows.append(jnp.dot(a35q_ref[i * blk:(i + 1) * blk, k0:k1],
                            xq[k0:k1, :], preferred_element_type=jnp.float32))
    c35col = jnp.concatenate(rows, axis=0) if nb > 1 else rows[0]

    u16 = _su(xf, 16)
    d16 = _sd(xf, 16)
    c31col = c35col - (u16 + _su(u16, 1) + d16 + _sd(d16, 1))

    c35b = c35col.astype(jnp.bfloat16)
    m31 = jnp.where(c31col > 0.5, 1.0, 0.0).astype(_F8)
    for j in range(nb):
        cs = slice(j * blk, (j + 1) * blk)
        k0, k1 = _kspan(j, w, r17) if nb > 1 else (0, w)
        rpiece = jnp.dot(c35b[:, k0:k1], a35b_ref[k0:k1, cs],
                         preferred_element_type=jnp.float32)
        rsm_ref[g, :, cs] = rpiece * (1.0 / (_RSM_K * _RSM_K))
        k0, k1 = _kspan(j, w, r15) if nb > 1 else (0, w)
        zpiece = jnp.dot(m31[:, k0:k1], a31q_ref[k0:k1, cs],
                         preferred_element_type=jnp.float32)
        pfm_ref[g, :, cs] = jnp.where(
            xf[:, cs] > 0.5, 1,
            jnp.where(zpiece > 0.5, 0, 2)).astype(jnp.int32)


def _make_body(imgs_per_step):
    def _body(x_ref, a35b_ref, a35q_ref, a31q_ref, rsm_ref, pfm_ref):
        for g in range(imgs_per_step):
            _one_image(x_ref[g], a35b_ref, a35q_ref, a31q_ref,
                       rsm_ref, pfm_ref, g)
    return _body


def _band(n, r, dtype):
    i = jnp.arange(n)
    return (jnp.abs(i[:, None] - i[None, :]) <= r).astype(dtype)


def kernel(masks):
    b, _, h, w = masks.shape
    x = masks.reshape(b, h, w)
    a35b = _band(w, _RSM_K // 2, jnp.bfloat16)
    a35q = _band(w, _RSM_K // 2, _F8)
    a31q = _band(w, _PFM_K // 2, _F8)
    g = 1
    rsm, pfm = pl.pallas_call(
        _make_body(g),
        grid=(b // g,),
        in_specs=[
            pl.BlockSpec((g, h, w), lambda i: (i, 0, 0)),
            pl.BlockSpec((w, w), lambda i: (0, 0)),
            pl.BlockSpec((w, w), lambda i: (0, 0)),
            pl.BlockSpec((w, w), lambda i: (0, 0)),
        ],
        out_specs=[
            pl.BlockSpec((g, h, w), lambda i: (i, 0, 0)),
            pl.BlockSpec((g, h, w), lambda i: (i, 0, 0)),
        ],
        out_shape=[
            jax.ShapeDtypeStruct((b, h, w), jnp.float32),
            jax.ShapeDtypeStruct((b, h, w), jnp.int32),
        ],
        compiler_params=pltpu.CompilerParams(
            dimension_semantics=("parallel",),
            vmem_limit_bytes=56 * 1024 * 1024,
        ),
        name="label_generator",
    )(x, a35b, a35q, a31q)
    return rsm.reshape(b, 1, h, w), pfm


# speedup vs baseline: 1.1271x; 1.1271x over previous
import jax
import jax.numpy as jnp
from jax.experimental import pallas as pl
from jax.experimental.pallas import tpu as pltpu

_RSM_K = 35
_PFM_K = 31
_F8 = jnp.float8_e4m3fn
_BLK = 256


def _su(x, d):
    return jnp.concatenate([x[d:, :], jnp.zeros((d, x.shape[1]), x.dtype)], axis=0)


def _sd(x, d):
    return jnp.concatenate([jnp.zeros((d, x.shape[1]), x.dtype), x[:-d, :]], axis=0)


def _kspan(j, w, r):
    k0 = max(0, ((j * _BLK - r) // 128) * 128)
    k1 = min(w, ((j * _BLK + _BLK + r + 127) // 128) * 128)
    return k0, k1


def _one_image(xf, a35b_ref, a35q_ref, a31q_ref, rsm_ref, pfm_ref, g):
    w = xf.shape[1]
    nb = w // _BLK if w % _BLK == 0 else 1
    blk = w // nb
    r17, r15 = _RSM_K // 2, _PFM_K // 2

    xq = xf.astype(_F8)
    rows = []
    for i in range(nb):
        k0, k1 = _kspan(i, w, r17) if nb > 1 else (0, w)
        rows.append(jnp.dot(a35q_ref[i * blk:(i + 1) * blk, k0:k1],
                            xq[k0:k1, :], preferred_element_type=jnp.float32))
    c35col = jnp.concatenate(rows, axis=0) if nb > 1 else rows[0]

    rows31 = []
    for i in range(nb):
        k0, k1 = _kspan(i, w, r15) if nb > 1 else (0, w)
        rows31.append(jnp.dot(a31q_ref[i * blk:(i + 1) * blk, k0:k1],
                              xq[k0:k1, :], preferred_element_type=jnp.float32))
    c31col = jnp.concatenate(rows31, axis=0) if nb > 1 else rows31[0]

    c35b = c35col.astype(jnp.bfloat16)
    m31 = jnp.where(c31col > 0.5, 1.0, 0.0).astype(_F8)
    for j in range(nb):
        cs = slice(j * blk, (j + 1) * blk)
        k0, k1 = _kspan(j, w, r17) if nb > 1 else (0, w)
        rpiece = jnp.dot(c35b[:, k0:k1], a35b_ref[k0:k1, cs],
                         preferred_element_type=jnp.float32)
        rsm_ref[g, :, cs] = rpiece * (1.0 / (_RSM_K * _RSM_K))
        k0, k1 = _kspan(j, w, r15) if nb > 1 else (0, w)
        zpiece = jnp.dot(m31[:, k0:k1], a31q_ref[k0:k1, cs],
                         preferred_element_type=jnp.float32)
        pfm_ref[g, :, cs] = jnp.where(
            xf[:, cs] > 0.5, 1,
            jnp.where(zpiece > 0.5, 0, 2)).astype(jnp.int32)


def _make_body(imgs_per_step):
    def _body(x_ref, a35b_ref, a35q_ref, a31q_ref, rsm_ref, pfm_ref):
        for g in range(imgs_per_step):
            _one_image(x_ref[g], a35b_ref, a35q_ref, a31q_ref,
                       rsm_ref, pfm_ref, g)
    return _body


def _band(n, r, dtype):
    i = jnp.arange(n)
    return (jnp.abs(i[:, None] - i[None, :]) <= r).astype(dtype)


def kernel(masks):
    b, _, h, w = masks.shape
    x = masks.reshape(b, h, w)
    a35b = _band(w, _RSM_K // 2, jnp.bfloat16)
    a35q = _band(w, _RSM_K // 2, _F8)
    a31q = _band(w, _PFM_K // 2, _F8)
    g = 2 if b % 2 == 0 else 1
    rsm, pfm = pl.pallas_call(
        _make_body(g),
        grid=(b // g,),
        in_specs=[
            pl.BlockSpec((g, h, w), lambda i: (i, 0, 0)),
            pl.BlockSpec((w, w), lambda i: (0, 0)),
            pl.BlockSpec((w, w), lambda i: (0, 0)),
            pl.BlockSpec((w, w), lambda i: (0, 0)),
        ],
        out_specs=[
            pl.BlockSpec((g, h, w), lambda i: (i, 0, 0)),
            pl.BlockSpec((g, h, w), lambda i: (i, 0, 0)),
        ],
        out_shape=[
            jax.ShapeDtypeStruct((b, h, w), jnp.float32),
            jax.ShapeDtypeStruct((b, h, w), jnp.int32),
        ],
        compiler_params=pltpu.CompilerParams(
            dimension_semantics=("parallel",),
            vmem_limit_bytes=56 * 1024 * 1024,
        ),
        name="label_generator",
    )(x, a35b, a35q, a31q)
    return rsm.reshape(b, 1, h, w), pfm


# speedup vs baseline: 1.1688x; 1.0370x over previous
import jax
import jax.numpy as jnp
from jax.experimental import pallas as pl
from jax.experimental.pallas import tpu as pltpu

_RSM_K = 35
_PFM_K = 31
_F8 = jnp.float8_e4m3fn
_BLK = 256


def _kspan(j, w, r):
    k0 = max(0, ((j * _BLK - r) // 128) * 128)
    k1 = min(w, ((j * _BLK + _BLK + r + 127) // 128) * 128)
    return k0, k1


def _one_image(xf, a35b_ref, a35q_ref, a31q_ref, rsm_ref, pfm_ref, g):
    w = xf.shape[1]
    nb = w // _BLK if w % _BLK == 0 else 1
    blk = w // nb
    r17, r15 = _RSM_K // 2, _PFM_K // 2

    xq = xf.astype(_F8)
    rows35, rows31 = [], []
    for i in range(nb):
        rs = slice(i * blk, (i + 1) * blk)
        k0, k1 = _kspan(i, w, r17) if nb > 1 else (0, w)
        rows35.append(jnp.dot(a35q_ref[rs, k0:k1], xq[k0:k1, :],
                              preferred_element_type=jnp.float32))
        k0, k1 = _kspan(i, w, r15) if nb > 1 else (0, w)
        rows31.append(jnp.dot(a31q_ref[rs, k0:k1], xq[k0:k1, :],
                              preferred_element_type=jnp.float32))
    c35col = jnp.concatenate(rows35, axis=0) if nb > 1 else rows35[0]
    c31col = jnp.concatenate(rows31, axis=0) if nb > 1 else rows31[0]

    c35b = c35col.astype(jnp.bfloat16)
    m31 = jnp.where(c31col > 0.5, 1.0, 0.0).astype(_F8)
    for j in range(nb):
        cs = slice(j * blk, (j + 1) * blk)
        k0, k1 = _kspan(j, w, r17) if nb > 1 else (0, w)
        rpiece = jnp.dot(c35b[:, k0:k1], a35b_ref[k0:k1, cs],
                         preferred_element_type=jnp.float32)
        rsm_ref[g, :, cs] = rpiece * (1.0 / (_RSM_K * _RSM_K))
        k0, k1 = _kspan(j, w, r15) if nb > 1 else (0, w)
        zpiece = jnp.dot(m31[:, k0:k1], a31q_ref[k0:k1, cs],
                         preferred_element_type=jnp.float32)
        pfm_ref[g, :, cs] = jnp.where(
            xf[:, cs] > 0.5, 1,
            jnp.where(zpiece > 0.5, 0, 2)).astype(jnp.int32)


def _make_body(imgs_per_step):
    def _body(x_ref, rsm_ref, pfm_ref, a35b_s, a35q_s, a31q_s):
        w = a35b_s.shape[0]

        @pl.when(pl.program_id(0) == 0)
        def _init():
            rows = jax.lax.broadcasted_iota(jnp.int32, (w, w), 0)
            cols = jax.lax.broadcasted_iota(jnp.int32, (w, w), 1)
            d = jnp.abs(rows - cols)
            in35 = d <= _RSM_K // 2
            a35b_s[...] = jnp.where(in35, 1.0, 0.0).astype(jnp.bfloat16)
            a35q_s[...] = jnp.where(in35, 1.0, 0.0).astype(_F8)
            a31q_s[...] = jnp.where(d <= _PFM_K // 2, 1.0, 0.0).astype(_F8)

        for g in range(imgs_per_step):
            _one_image(x_ref[g], a35b_s, a35q_s, a31q_s,
                       rsm_ref, pfm_ref, g)
    return _body


def kernel(masks):
    b, _, h, w = masks.shape
    x = masks.reshape(b, h, w)
    g = 2 if b % 2 == 0 else 1
    rsm, pfm = pl.pallas_call(
        _make_body(g),
        grid=(b // g,),
        in_specs=[
            pl.BlockSpec((g, h, w), lambda i: (i, 0, 0)),
        ],
        out_specs=[
            pl.BlockSpec((g, h, w), lambda i: (i, 0, 0)),
            pl.BlockSpec((g, h, w), lambda i: (i, 0, 0)),
        ],
        out_shape=[
            jax.ShapeDtypeStruct((b, h, w), jnp.float32),
            jax.ShapeDtypeStruct((b, h, w), jnp.int32),
        ],
        scratch_shapes=[
            pltpu.VMEM((w, w), jnp.bfloat16),
            pltpu.VMEM((w, w), _F8),
            pltpu.VMEM((w, w), _F8),
        ],
        compiler_params=pltpu.CompilerParams(
            dimension_semantics=("arbitrary",),
            vmem_limit_bytes=56 * 1024 * 1024,
        ),
        name="label_generator",
    )(x)
    return rsm.reshape(b, 1, h, w), pfm
